# baseline (speedup 1.0000x reference)
import functools

import numpy as np

import jax
import jax.numpy as jnp
from jax import lax
from jax.experimental import pallas as pl
from jax.experimental.pallas import tpu as pltpu
from jax.experimental.pallas import tpu_sc as plsc

_B = 16384
_NS = 4
_S = _B // _NS
_F = 26
_D = 32
_FD = _F * _D
_SLOTS = 28
_NK = _SLOTS // 4
_FP = _NK * 128
_H1, _H2 = 256, 128

_NW = 32
_SPT = _S // _NW
_SPG = 32
_NG = _SPT // _SPG
_CW = 128
_NCH = _NG * _NK
_GROWS = _NK * _CW

_LCW = 128
_LNCH = _S * _F // _NW // _LCW

_mesh = plsc.VectorSubcoreMesh(core_axis_name="c", subcore_axis_name="s")


def _sc_gather_body(xe_hbm, xl_hbm, table_hbm, lin_hbm,
                    emb_out, lin_out,
                    idx_e, idx_l, eb0, eb1, linv, sem_e, sem_l, sem_w):
    c = lax.axis_index("c")
    s = lax.axis_index("s")
    wid = s * 2 + c
    gbase = wid * _SPT * 4
    lbase = wid * _LNCH

    pltpu.sync_copy(xe_hbm.at[wid], idx_e)
    pltpu.sync_copy(xl_hbm.at[wid], idx_l)

    lin_cps = [
        pltpu.async_copy(lin_hbm.at[idx_l.at[k]], linv.at[k], sem_l)
        for k in range(_LNCH)
    ]

    ebufs = (eb0, eb1)

    def fire_gather(g):
        return [
            pltpu.async_copy(
                table_hbm.at[idx_e.at[g * _NK + k]],
                ebufs[g % 2].at[pl.ds(k * _CW, _CW)], sem_e)
            for k in range(_NK)
        ]

    def fire_write(g):
        buf = ebufs[g % 2]
        return [
            pltpu.async_copy(
                buf.at[pl.ds(k * _CW, _CW)],
                emb_out.at[k, pl.ds(gbase + g * _CW, _CW)], sem_w)
            for k in range(_NK)
        ]

    g_cp = {0: fire_gather(0)}
    w_cp = [None, None]
    for g in range(_NG):
        for cp in g_cp.pop(g):
            cp.wait()
        w_cp[g % 2] = fire_write(g)
        if g + 1 < _NG:
            if w_cp[(g + 1) % 2] is not None:
                for cp in w_cp[(g + 1) % 2]:
                    cp.wait()
            g_cp[g + 1] = fire_gather(g + 1)
    for cp in lin_cps:
        cp.wait()
    pltpu.sync_copy(linv, lin_out.at[pl.ds(lbase, _LNCH)])
    for cps in w_cp:
        if cps is not None:
            for cp in cps:
                cp.wait()


_sc_gather = functools.partial(
    pl.kernel,
    mesh=_mesh,
    compiler_params=pltpu.CompilerParams(use_tc_tiling_on_sc=False),
    out_type=[
        jax.ShapeDtypeStruct((_NK, _S * 4, _D), jnp.float32),
        jax.ShapeDtypeStruct((_S * _F // _LCW, _LCW), jnp.float32),
    ],
    scratch_types=[
        pltpu.VMEM((_NCH, _CW), jnp.int32),
        pltpu.VMEM((_LNCH, _LCW), jnp.int32),
        pltpu.VMEM((_GROWS, _D), jnp.float32),
        pltpu.VMEM((_GROWS, _D), jnp.float32),
        pltpu.VMEM((_LNCH, _LCW), jnp.float32),
        pltpu.SemaphoreType.DMA,
        pltpu.SemaphoreType.DMA,
        pltpu.SemaphoreType.DMA,
    ],
)(_sc_gather_body)


_BB = 512


def _tc_body(emb_ref, lin_ref, w1_ref, b1_ref, w2_ref, b2_ref,
             w3_ref, b3_ref, s_ref, out_ref):
    eks = [emb_ref[k] for k in range(_NK)]
    lane = lax.broadcasted_iota(jnp.int32, (1, 128), 1)
    eks[6] = jnp.where(lane < 64, eks[6], 0.0)
    e896 = jnp.concatenate(eks, axis=1)

    h1 = jnp.maximum(
        jnp.dot(e896, w1_ref[...], preferred_element_type=jnp.float32)
        + b1_ref[...], 0.0)
    h2 = jnp.maximum(
        jnp.dot(h1, w2_ref[...], preferred_element_type=jnp.float32)
        + b2_ref[...], 0.0)
    dnn = jnp.sum(h2 * w3_ref[...], axis=1, keepdims=True)

    fs = jnp.dot(e896, s_ref[...], preferred_element_type=jnp.float32)
    fm = 0.5 * (jnp.sum(fs * fs, axis=1, keepdims=True)
                - jnp.sum(e896 * e896, axis=1, keepdims=True))

    linsum = jnp.sum(lin_ref[...], axis=0)[:, None]
    z = linsum + fm + dnn + b3_ref[...]
    out_ref[...] = 1.0 / (1.0 + jnp.exp(-z))


_tc_call = pl.pallas_call(
    _tc_body,
    grid=(_S // _BB,),
    in_specs=[
        pl.BlockSpec((_NK, _BB, 128), lambda i: (0, i, 0)),
        pl.BlockSpec((_F, _BB), lambda i: (0, i)),
        pl.BlockSpec((_FP, _H1), lambda i: (0, 0)),
        pl.BlockSpec((1, _H1), lambda i: (0, 0)),
        pl.BlockSpec((_H1, _H2), lambda i: (0, 0)),
        pl.BlockSpec((1, _H2), lambda i: (0, 0)),
        pl.BlockSpec((1, _H2), lambda i: (0, 0)),
        pl.BlockSpec((1, 1), lambda i: (0, 0)),
        pl.BlockSpec((_FP, _H2), lambda i: (0, 0)),
    ],
    out_specs=pl.BlockSpec((_BB, 1), lambda i: (i, 0)),
    out_shape=jax.ShapeDtypeStruct((_S, 1), jnp.float32),
)

_S_MAT = np.zeros((_FP, _H2), np.float32)
for _p in range(_FD):
    _S_MAT[_p, _p % _D] = 1.0


def kernel(x, table, lin_table, W1, b1, W2, b2, W3, b3):
    lin1 = lin_table.reshape(-1)
    w1p = jnp.concatenate(
        [W1, jnp.zeros((_FP - _FD, _H1), jnp.float32)], axis=0)
    smat = jnp.asarray(_S_MAT)
    b1r, b2r = b1.reshape(1, -1), b2.reshape(1, -1)
    w3r, b3r = W3.reshape(1, -1), b3.reshape(1, 1)

    gathered = []
    for h in range(_NS):
        xs = x[h * _S:(h + 1) * _S]
        xpad = jnp.concatenate([xs, xs[:, : _SLOTS - _F]], axis=1)
        xe = (xpad.reshape(_S // _SPG, _SPG, _NK, 4)
              .transpose(0, 2, 1, 3)
              .reshape(_NW, _NCH, _CW))
        xl = xs.T.reshape(_NW, _LNCH, _LCW)
        gathered.append(_sc_gather(xe, xl, table, lin1))

    outs = []
    for h in range(_NS):
        emb_p, lin_flat = gathered[h]
        e3 = emb_p.reshape(_NK, _S, 128)
        ling = lin_flat.reshape(_F, _S)
        outs.append(_tc_call(e3, ling, w1p, b1r, W2, b2r, w3r, b3r, smat))
    return jnp.concatenate(outs, axis=0)


# speedup vs baseline: 1.0588x; 1.0588x over previous
import functools

import numpy as np

import jax
import jax.numpy as jnp
from jax import lax
from jax.experimental import pallas as pl
from jax.experimental.pallas import tpu as pltpu
from jax.experimental.pallas import tpu_sc as plsc

_B = 16384
_NS = 2
_S = _B // _NS
_F = 26
_D = 32
_FD = _F * _D
_SLOTS = 28
_NK = _SLOTS // 4
_FP = _NK * 128
_H1, _H2 = 256, 128

_NW = 32
_SPT = _S // _NW
_SPG = 32
_NG = _SPT // _SPG
_CW = 128
_NCH = _NG * _NK
_GROWS = _NK * _CW

_LCW = 128
_LNCH = _S * _F // _NW // _LCW

_mesh = plsc.VectorSubcoreMesh(core_axis_name="c", subcore_axis_name="s")


def _sc_gather_body(xe_hbm, xl_hbm, table_hbm, lin_hbm,
                    emb_out, lin_out,
                    idx_e, idx_l, eb0, eb1, linv, sem_e, sem_l, sem_w):
    c = lax.axis_index("c")
    s = lax.axis_index("s")
    wid = s * 2 + c
    gbase = wid * _SPT * 4
    lbase = wid * _LNCH

    pltpu.sync_copy(xe_hbm.at[wid], idx_e)
    pltpu.sync_copy(xl_hbm.at[wid], idx_l)

    lin_cps = [
        pltpu.async_copy(lin_hbm.at[idx_l.at[k]], linv.at[k], sem_l)
        for k in range(_LNCH)
    ]

    ebufs = (eb0, eb1)

    def fire_gather(g):
        return [
            pltpu.async_copy(
                table_hbm.at[idx_e.at[g * _NK + k]],
                ebufs[g % 2].at[pl.ds(k * _CW, _CW)], sem_e)
            for k in range(_NK)
        ]

    def fire_write(g):
        buf = ebufs[g % 2]
        return [
            pltpu.async_copy(
                buf.at[pl.ds(k * _CW, _CW)],
                emb_out.at[k, pl.ds(gbase + g * _CW, _CW)], sem_w)
            for k in range(_NK)
        ]

    g_cp = {0: fire_gather(0)}
    w_cp = [None, None]
    for g in range(_NG):
        for cp in g_cp.pop(g):
            cp.wait()
        w_cp[g % 2] = fire_write(g)
        if g + 1 < _NG:
            if w_cp[(g + 1) % 2] is not None:
                for cp in w_cp[(g + 1) % 2]:
                    cp.wait()
            g_cp[g + 1] = fire_gather(g + 1)
    for cp in lin_cps:
        cp.wait()
    pltpu.sync_copy(linv, lin_out.at[pl.ds(lbase, _LNCH)])
    for cps in w_cp:
        if cps is not None:
            for cp in cps:
                cp.wait()


_sc_gather = functools.partial(
    pl.kernel,
    mesh=_mesh,
    compiler_params=pltpu.CompilerParams(use_tc_tiling_on_sc=False),
    out_type=[
        jax.ShapeDtypeStruct((_NK, _S * 4, _D), jnp.float32),
        jax.ShapeDtypeStruct((_S * _F // _LCW, _LCW), jnp.float32),
    ],
    scratch_types=[
        pltpu.VMEM((_NCH, _CW), jnp.int32),
        pltpu.VMEM((_LNCH, _LCW), jnp.int32),
        pltpu.VMEM((_GROWS, _D), jnp.float32),
        pltpu.VMEM((_GROWS, _D), jnp.float32),
        pltpu.VMEM((_LNCH, _LCW), jnp.float32),
        pltpu.SemaphoreType.DMA,
        pltpu.SemaphoreType.DMA,
        pltpu.SemaphoreType.DMA,
    ],
)(_sc_gather_body)


_BB = 1024


def _tc_body(emb_ref, lin_ref, w1_ref, b1_ref, w2_ref, b2_ref,
             w3_ref, b3_ref, s_ref, out_ref):
    eks = [emb_ref[k] for k in range(_NK)]
    lane = lax.broadcasted_iota(jnp.int32, (1, 128), 1)
    eks[6] = jnp.where(lane < 64, eks[6], 0.0)
    e896 = jnp.concatenate(eks, axis=1)

    h1 = jnp.maximum(
        jnp.dot(e896, w1_ref[...], preferred_element_type=jnp.float32)
        + b1_ref[...], 0.0)
    h2 = jnp.maximum(
        jnp.dot(h1, w2_ref[...], preferred_element_type=jnp.float32)
        + b2_ref[...], 0.0)
    dnn = jnp.sum(h2 * w3_ref[...], axis=1, keepdims=True)

    fs = jnp.dot(e896, s_ref[...], preferred_element_type=jnp.float32)
    fm = 0.5 * (jnp.sum(fs * fs, axis=1, keepdims=True)
                - jnp.sum(e896 * e896, axis=1, keepdims=True))

    linsum = jnp.sum(lin_ref[...], axis=0)[:, None]
    z = linsum + fm + dnn + b3_ref[...]
    out_ref[...] = 1.0 / (1.0 + jnp.exp(-z))


_tc_call = pl.pallas_call(
    _tc_body,
    grid=(_S // _BB,),
    in_specs=[
        pl.BlockSpec((_NK, _BB, 128), lambda i: (0, i, 0)),
        pl.BlockSpec((_F, _BB), lambda i: (0, i)),
        pl.BlockSpec((_FP, _H1), lambda i: (0, 0)),
        pl.BlockSpec((1, _H1), lambda i: (0, 0)),
        pl.BlockSpec((_H1, _H2), lambda i: (0, 0)),
        pl.BlockSpec((1, _H2), lambda i: (0, 0)),
        pl.BlockSpec((1, _H2), lambda i: (0, 0)),
        pl.BlockSpec((1, 1), lambda i: (0, 0)),
        pl.BlockSpec((_FP, _H2), lambda i: (0, 0)),
    ],
    out_specs=pl.BlockSpec((_BB, 1), lambda i: (i, 0)),
    out_shape=jax.ShapeDtypeStruct((_S, 1), jnp.float32),
)

_S_MAT = np.zeros((_FP, _H2), np.float32)
for _p in range(_FD):
    _S_MAT[_p, _p % _D] = 1.0


def kernel(x, table, lin_table, W1, b1, W2, b2, W3, b3):
    lin1 = lin_table.reshape(-1)
    w1p = jnp.concatenate(
        [W1, jnp.zeros((_FP - _FD, _H1), jnp.float32)], axis=0)
    smat = jnp.asarray(_S_MAT)
    b1r, b2r = b1.reshape(1, -1), b2.reshape(1, -1)
    w3r, b3r = W3.reshape(1, -1), b3.reshape(1, 1)

    gathered = []
    for h in range(_NS):
        xs = x[h * _S:(h + 1) * _S]
        xpad = jnp.concatenate([xs, xs[:, : _SLOTS - _F]], axis=1)
        xe = (xpad.reshape(_S // _SPG, _SPG, _NK, 4)
              .transpose(0, 2, 1, 3)
              .reshape(_NW, _NCH, _CW))
        xl = xs.T.reshape(_NW, _LNCH, _LCW)
        gathered.append(_sc_gather(xe, xl, table, lin1))

    outs = []
    for h in range(_NS):
        emb_p, lin_flat = gathered[h]
        e3 = emb_p.reshape(_NK, _S, 128)
        ling = lin_flat.reshape(_F, _S)
        outs.append(_tc_call(e3, ling, w1p, b1r, W2, b2r, w3r, b3r, smat))
    return jnp.concatenate(outs, axis=0)
